# Initial kernel scaffold; baseline (speedup 1.0000x reference)
import jax
import jax.numpy as jnp
from jax.experimental import pallas as pl


def kernel(inputs, duty_cycle):
    raise NotImplementedError("write your pallas kernel here")


# speedup vs baseline: 9.4463x; 9.4463x over previous
import functools

import jax
import jax.numpy as jnp
from jax.experimental import pallas as pl
from jax.experimental.pallas import tpu as pltpu

_K = 1024
_BETA = 1.0
_UNITS = 32768
_TARGET_DUTY = _K / _UNITS
_ROWS_PER_BLOCK = 8
_INT_MIN = -2147483648
_INT_MAX = 2147483647


def _sortable_key(x):
    i = jax.lax.bitcast_convert_type(x, jnp.int32)
    flip = jax.lax.shift_right_arithmetic(i, 31)
    return jnp.bitwise_xor(i, jnp.bitwise_and(flip, jnp.int32(0x7FFFFFFF)))


def _kwinner_block(in_ref, duty_ref, out_ref):
    x = in_ref[...]
    duty = duty_ref[...]
    boost = jnp.exp(_BETA * (_TARGET_DUTY - duty))
    key = _sortable_key(x * boost)

    def count_ge(t):
        m = (key >= t).astype(jnp.int32)
        return jnp.sum(m, axis=1, keepdims=True)

    zero = jnp.zeros((_ROWS_PER_BLOCK, 1), jnp.int32)
    c0 = count_ge(zero)
    ge = c0 >= _K
    lo = jnp.where(ge, jnp.int32(0), jnp.int32(_INT_MIN))
    hi = jnp.where(ge, jnp.int32(_INT_MAX), jnp.int32(-1))

    def body(_, carry):
        lo, hi = carry
        diff = hi - lo
        mid = lo + jax.lax.shift_right_logical(diff, 1) \
                 + jnp.bitwise_and(diff, jnp.int32(1))
        ok = count_ge(mid) >= _K
        return jnp.where(ok, mid, lo), jnp.where(ok, hi, mid - 1)

    lo, hi = jax.lax.fori_loop(0, 31, body, (lo, hi))
    out_ref[...] = jnp.where(key >= lo, x, jnp.float32(0.0))


@jax.jit
def kernel(inputs, duty_cycle):
    b, n = inputs.shape
    duty2d = duty_cycle.reshape(1, n)
    grid = (b // _ROWS_PER_BLOCK,)
    return pl.pallas_call(
        _kwinner_block,
        grid=grid,
        in_specs=[
            pl.BlockSpec((_ROWS_PER_BLOCK, n), lambda i: (i, 0)),
            pl.BlockSpec((1, n), lambda i: (0, 0)),
        ],
        out_specs=pl.BlockSpec((_ROWS_PER_BLOCK, n), lambda i: (i, 0)),
        out_shape=jax.ShapeDtypeStruct((b, n), jnp.float32),
    )(inputs, duty2d)


# speedup vs baseline: 11.1637x; 1.1818x over previous
import functools

import jax
import jax.numpy as jnp
from jax.experimental import pallas as pl
from jax.experimental.pallas import tpu as pltpu

_K = 1024
_BETA = 1.0
_UNITS = 32768
_TARGET_DUTY = _K / _UNITS
_ROWS_PER_BLOCK = 8
_INT_MIN = -2147483648
_INT_MAX = 2147483647


def _sortable_key(x):
    i = jax.lax.bitcast_convert_type(x, jnp.int32)
    flip = jax.lax.shift_right_arithmetic(i, 31)
    return jnp.bitwise_xor(i, jnp.bitwise_and(flip, jnp.int32(0x7FFFFFFF)))


def _kwinner_block(in_ref, duty_ref, out_ref):
    x = in_ref[...]
    duty = duty_ref[...]
    boost = jnp.exp(_BETA * (_TARGET_DUTY - duty))
    key = _sortable_key(x * boost)

    def count_ge(t):
        m = (key >= t).astype(jnp.int32)
        return jnp.sum(m, axis=1, keepdims=True)

    zero = jnp.zeros((_ROWS_PER_BLOCK, 1), jnp.int32)
    c0 = count_ge(zero)
    ge = c0 >= _K
    lo = jnp.where(ge, jnp.int32(0), jnp.int32(_INT_MIN))
    hi = jnp.where(ge, jnp.int32(_INT_MAX), jnp.int32(-1))
    c_lo = jnp.where(ge, c0, jnp.int32(_UNITS))

    def cond(carry):
        i, _, _, c_lo = carry
        return jnp.logical_and(i < 31, jnp.any(c_lo != _K))

    def body(carry):
        i, lo, hi, c_lo = carry
        diff = hi - lo
        mid = lo + jax.lax.shift_right_logical(diff, 1) \
                 + jnp.bitwise_and(diff, jnp.int32(1))
        c = count_ge(mid)
        ok = c >= _K
        return (i + 1, jnp.where(ok, mid, lo), jnp.where(ok, hi, mid - 1),
                jnp.where(ok, c, c_lo))

    _, lo, _, _ = jax.lax.while_loop(
        cond, body, (jnp.int32(0), lo, hi, c_lo))
    t = jnp.min(jnp.where(key >= lo, key, jnp.int32(_INT_MAX)),
                axis=1, keepdims=True)
    out_ref[...] = jnp.where(key >= t, x, jnp.float32(0.0))


@jax.jit
def kernel(inputs, duty_cycle):
    b, n = inputs.shape
    duty2d = duty_cycle.reshape(1, n)
    grid = (b // _ROWS_PER_BLOCK,)
    return pl.pallas_call(
        _kwinner_block,
        grid=grid,
        in_specs=[
            pl.BlockSpec((_ROWS_PER_BLOCK, n), lambda i: (i, 0)),
            pl.BlockSpec((1, n), lambda i: (0, 0)),
        ],
        out_specs=pl.BlockSpec((_ROWS_PER_BLOCK, n), lambda i: (i, 0)),
        out_shape=jax.ShapeDtypeStruct((b, n), jnp.float32),
    )(inputs, duty2d)
